# baseline (speedup 1.0000x reference)
import functools

import jax
import jax.numpy as jnp
from jax import lax
from jax.experimental import pallas as pl
from jax.experimental.pallas import tpu as pltpu
from jax.experimental.pallas import tpu_sc as plsc

N = 625
NC = 640
NR = 626
FLAT = NR * NC
E = 20000
EP = 20480
NCORES = 1
NTILES = 16
EPT = EP // (NCORES * NTILES)
TAIL = E - (NTILES - 1) * EPT
IDX_ROWS = EPT // 128
CHUNK = FLAT // NTILES
ZB = 3200
D_EMB = 64
EPS = 1e-5


def _sc_body(edge_hbm, out_hbm, ev_v, idx_v, one_v, stage_v, zb_v, sem, acc):
    tid = lax.axis_index("s")
    ebase = pl.multiple_of(tid * EPT, 128)
    cbase = pl.multiple_of(tid * CHUNK, 8)

    def _zb(m, _):
        zb_v[pl.ds(m * 16, 16)] = jnp.zeros((16,), jnp.float32)
        return 0
    lax.fori_loop(0, ZB // 16, _zb, 0)
    cps = []
    off = 0
    while off < CHUNK:
        sz = min(ZB, CHUNK - off)
        cps.append(pltpu.async_copy(zb_v.at[pl.ds(0, sz)],
                                    acc.at[pl.ds(cbase + off, sz)], sem))
        off += sz

    pltpu.sync_copy(edge_hbm.at[:, pl.ds(ebase, EPT)], ev_v)

    @pl.when(tid == NTILES - 1)
    def _fix_tail():
        def _pad(m, _):
            ev_v[1, pl.ds(TAIL + m * 16, 16)] = jnp.full((16,), N, jnp.int32)
            return 0
        lax.fori_loop(0, (EPT - TAIL) // 16, _pad, 0)

    def _ones(j, _):
        one_v[0, pl.ds(j * 16, 16)] = jnp.full((16,), 1.0, jnp.float32)
        return 0
    lax.fori_loop(0, 8, _ones, 0)
    for r in range(IDX_ROWS):
        def _ix(j, _, r=r):
            eoff = r * 128 + j * 16
            d16 = ev_v[1, pl.ds(eoff, 16)]
            s16 = ev_v[0, pl.ds(eoff, 16)]
            idx_v[r, pl.ds(j * 16, 16)] = d16 * NC + s16
            return 0
        lax.fori_loop(0, 8, _ix, 0)

    for cp in cps:
        cp.wait()
    plsc.subcore_barrier()

    cps = [pltpu.async_copy(one_v.at[0], acc.at[idx_v.at[j]], sem, add=True)
           for j in range(IDX_ROWS)]
    for cp in cps:
        cp.wait()

    plsc.subcore_barrier()

    h1 = 12528
    h2 = CHUNK - h1
    pltpu.sync_copy(acc.at[pl.ds(cbase, h1)], stage_v.at[pl.ds(0, h1)])
    cp1 = pltpu.async_copy(stage_v.at[pl.ds(0, h1)],
                           out_hbm.at[pl.ds(cbase, h1)], sem)
    pltpu.sync_copy(acc.at[pl.ds(cbase + h1, h2)], stage_v.at[pl.ds(h1, h2)])
    cp1.wait()
    pltpu.sync_copy(stage_v.at[pl.ds(h1, h2)], out_hbm.at[pl.ds(cbase + h1, h2)])


def _sc_build_adj(edge_p):
    mesh = plsc.VectorSubcoreMesh(core_axis_name="c", subcore_axis_name="s",
                                  num_cores=NCORES)
    f = functools.partial(
        pl.kernel,
        mesh=mesh,
        out_type=jax.ShapeDtypeStruct((FLAT,), jnp.float32),
        scratch_types=[
            pltpu.VMEM((2, EPT), jnp.int32),
            pltpu.VMEM((IDX_ROWS, 128), jnp.int32),
            pltpu.VMEM((1, 128), jnp.float32),
            pltpu.VMEM((CHUNK,), jnp.float32),
            pltpu.VMEM((ZB,), jnp.float32),
            pltpu.SemaphoreType.DMA,
            pltpu.VMEM_SHARED((FLAT,), jnp.float32),
        ],
    )(_sc_body)
    return f(edge_p)


def _dot(a, b):
    return lax.dot_general(a, b, (((1,), (0,)), ((), ())),
                           preferred_element_type=jnp.float32)


def _agg(A, y, dinv):
    ys = y * dinv
    m = lax.dot_general(A.astype(jnp.bfloat16), ys.astype(jnp.bfloat16),
                        (((1,), (0,)), ((), ())),
                        preferred_element_type=jnp.float32)
    return (m + ys) * dinv


def _tile(v, bsz):
    return jnp.concatenate([v] * bsz, axis=1)


def _fold(s, bsz):
    acc = lax.slice(s, (0, 0), (1, D_EMB))
    for g in range(1, bsz):
        acc = acc + lax.slice(s, (0, g * D_EMB), (1, (g + 1) * D_EMB))
    return acc


def _bn_nm(h, w, b, bsz):
    cnt = float(bsz * N)
    s = jnp.sum(h, axis=0, keepdims=True)
    s2 = jnp.sum(h * h, axis=0, keepdims=True)
    mu = _fold(s, bsz) / cnt
    m2 = _fold(s2, bsz) / cnt
    var = m2 - mu * mu
    scale = lax.rsqrt(var + EPS) * w[None, :]
    shift = b[None, :] - mu * scale
    return h * _tile(scale, bsz) + _tile(shift, bsz)


def _tc_body(bsz, feat, a_ref, x_ref, tb_ref, w1_ref, b1_ref, w2_ref, b2_ref,
             w3_ref, b3_ref, g1_ref, be1_ref, g2_ref, be2_ref, o_ref, y_ref):
    g = pl.program_id(0)
    y_ref[g] = _dot(x_ref[0], w1_ref[...])

    @pl.when(g == bsz - 1)
    def _tail():
        a = a_ref[:N, :]
        deg = jnp.sum(a, axis=1, keepdims=True) + 1.0
        dinv = lax.rsqrt(deg)
        A = a[:, :N]

        y = jnp.concatenate([y_ref[i] for i in range(bsz)], axis=1)
        h = _agg(A, y, dinv) + _tile(b1_ref[...][None, :], bsz)
        h = jnp.maximum(h, 0.0)
        h = _bn_nm(h, g1_ref[...], be1_ref[...], bsz)

        w2 = w2_ref[...]
        y = jnp.concatenate(
            [_dot(h[:, i * D_EMB:(i + 1) * D_EMB], w2) for i in range(bsz)],
            axis=1)
        h = _agg(A, y, dinv) + _tile(b2_ref[...][None, :], bsz)
        h = jnp.maximum(h, 0.0)
        h = _bn_nm(h, g2_ref[...], be2_ref[...], bsz)

        w3 = w3_ref[...]
        y = jnp.concatenate(
            [_dot(h[:, i * D_EMB:(i + 1) * D_EMB], w3) for i in range(bsz)],
            axis=1)
        h = _agg(A, y, dinv) + _tile(b3_ref[...][None, :], bsz)
        h = jnp.maximum(h, 0.0)
        o_ref[...] = h + (tb_ref[0, 0] - bsz).astype(jnp.float32)


def _tc_forward(a_mat, x3, tb, W1, b1, W2, b2, W3, b3, bn1_w, bn1_b, bn2_w,
                bn2_b, bsz):
    feat = x3.shape[-1]
    full = lambda g: (0, 0)
    return pl.pallas_call(
        functools.partial(_tc_body, bsz, feat),
        grid=(bsz,),
        in_specs=[
            pl.BlockSpec((NR, NC), full),
            pl.BlockSpec((1, N, feat), lambda g: (g, 0, 0)),
            pl.BlockSpec(memory_space=pltpu.SMEM),
            pl.BlockSpec((feat, D_EMB), full),
            pl.BlockSpec((D_EMB,), lambda g: (0,)),
            pl.BlockSpec((D_EMB, D_EMB), full),
            pl.BlockSpec((D_EMB,), lambda g: (0,)),
            pl.BlockSpec((D_EMB, D_EMB), full),
            pl.BlockSpec((D_EMB,), lambda g: (0,)),
            pl.BlockSpec((D_EMB,), lambda g: (0,)),
            pl.BlockSpec((D_EMB,), lambda g: (0,)),
            pl.BlockSpec((D_EMB,), lambda g: (0,)),
            pl.BlockSpec((D_EMB,), lambda g: (0,)),
        ],
        out_specs=pl.BlockSpec((N, bsz * D_EMB), full),
        out_shape=jax.ShapeDtypeStruct((N, bsz * D_EMB), jnp.float32),
        scratch_shapes=[pltpu.VMEM((bsz, N, D_EMB), jnp.float32)],
    )(a_mat, x3, tb, W1, b1, W2, b2, W3, b3, bn1_w, bn1_b, bn2_w, bn2_b)


def kernel(node_list, edge_index, true_batch_size, W1, b1, W2, b2, W3, b3,
           bn1_w, bn1_b, bn2_w, bn2_b):
    bsz, n_per_graph, feat = node_list.shape
    ei_p = jnp.pad(edge_index, ((0, 0), (0, EP - E)))
    a_mat = _sc_build_adj(ei_p).reshape(NR, NC)

    tb = jnp.asarray(true_batch_size, jnp.int32).reshape(1, 1)
    h = _tc_forward(a_mat, node_list, tb, W1, b1, W2, b2, W3, b3,
                    bn1_w, bn1_b, bn2_w, bn2_b, bsz)
    return h.reshape(n_per_graph, bsz, D_EMB).transpose(1, 0, 2)


# speedup vs baseline: 1.1959x; 1.1959x over previous
import functools

import jax
import jax.numpy as jnp
from jax import lax
from jax.experimental import pallas as pl
from jax.experimental.pallas import tpu as pltpu
from jax.experimental.pallas import tpu_sc as plsc

N = 625
NC = 640
NR = 626
FLAT = NR * NC
E = 20000
EP = 20480
NCORES = 1
NTILES = 16
EPT = EP // (NCORES * NTILES)
TAIL = E - (NTILES - 1) * EPT
IDX_ROWS = EPT // 128
CHUNK = FLAT // NTILES
ZB = 3200
D_EMB = 64
EPS = 1e-5


def _sc_body(edge_hbm, out_hbm, ev_v, idx_v, one_v, stage_v, zb_v, sem, acc):
    tid = lax.axis_index("s")
    ebase = pl.multiple_of(tid * EPT, 128)
    cbase = pl.multiple_of(tid * CHUNK, 8)

    def _zb(m, _):
        zb_v[pl.ds(m * 16, 16)] = jnp.zeros((16,), jnp.float32)
        return 0
    lax.fori_loop(0, ZB // 16, _zb, 0)
    cps = []
    off = 0
    while off < CHUNK:
        sz = min(ZB, CHUNK - off)
        cps.append(pltpu.async_copy(zb_v.at[pl.ds(0, sz)],
                                    acc.at[pl.ds(cbase + off, sz)], sem))
        off += sz

    pltpu.sync_copy(edge_hbm.at[:, pl.ds(ebase, EPT)], ev_v)

    @pl.when(tid == NTILES - 1)
    def _fix_tail():
        def _pad(m, _):
            ev_v[1, pl.ds(TAIL + m * 16, 16)] = jnp.full((16,), N, jnp.int32)
            return 0
        lax.fori_loop(0, (EPT - TAIL) // 16, _pad, 0)

    def _ones(j, _):
        one_v[0, pl.ds(j * 16, 16)] = jnp.full((16,), 1.0, jnp.float32)
        return 0
    lax.fori_loop(0, 8, _ones, 0)
    for r in range(IDX_ROWS):
        def _ix(j, _, r=r):
            eoff = r * 128 + j * 16
            d16 = ev_v[1, pl.ds(eoff, 16)]
            s16 = ev_v[0, pl.ds(eoff, 16)]
            idx_v[r, pl.ds(j * 16, 16)] = d16 * NC + s16
            return 0
        lax.fori_loop(0, 8, _ix, 0)

    for cp in cps:
        cp.wait()
    plsc.subcore_barrier()

    cps = [pltpu.async_copy(one_v.at[0], acc.at[idx_v.at[j]], sem, add=True)
           for j in range(IDX_ROWS)]
    for cp in cps:
        cp.wait()

    plsc.subcore_barrier()

    h1 = 12528
    h2 = CHUNK - h1
    pltpu.sync_copy(acc.at[pl.ds(cbase, h1)], stage_v.at[pl.ds(0, h1)])
    cp1 = pltpu.async_copy(stage_v.at[pl.ds(0, h1)],
                           out_hbm.at[pl.ds(cbase, h1)], sem)
    pltpu.sync_copy(acc.at[pl.ds(cbase + h1, h2)], stage_v.at[pl.ds(h1, h2)])
    cp1.wait()
    pltpu.sync_copy(stage_v.at[pl.ds(h1, h2)], out_hbm.at[pl.ds(cbase + h1, h2)])


def _sc_build_adj(edge_p):
    mesh = plsc.VectorSubcoreMesh(core_axis_name="c", subcore_axis_name="s",
                                  num_cores=NCORES)
    f = functools.partial(
        pl.kernel,
        mesh=mesh,
        out_type=jax.ShapeDtypeStruct((FLAT,), jnp.float32),
        scratch_types=[
            pltpu.VMEM((2, EPT), jnp.int32),
            pltpu.VMEM((IDX_ROWS, 128), jnp.int32),
            pltpu.VMEM((1, 128), jnp.float32),
            pltpu.VMEM((CHUNK,), jnp.float32),
            pltpu.VMEM((ZB,), jnp.float32),
            pltpu.SemaphoreType.DMA,
            pltpu.VMEM_SHARED((FLAT,), jnp.float32),
        ],
    )(_sc_body)
    return f(edge_p)


def _dot(a, b):
    return lax.dot_general(a, b, (((1,), (0,)), ((), ())),
                           preferred_element_type=jnp.float32)


def _agg(A, y, dinv):
    ys = y * dinv
    m = lax.dot_general(A.astype(jnp.bfloat16), ys.astype(jnp.bfloat16),
                        (((1,), (0,)), ((), ())),
                        preferred_element_type=jnp.float32)
    return (m + ys) * dinv


def _tile(v, bsz):
    return jnp.concatenate([v] * bsz, axis=1)


def _fold(s, bsz):
    acc = lax.slice(s, (0, 0), (1, D_EMB))
    for g in range(1, bsz):
        acc = acc + lax.slice(s, (0, g * D_EMB), (1, (g + 1) * D_EMB))
    return acc


def _bn_nm(h, w, b, bsz):
    cnt = float(bsz * N)
    s = jnp.sum(h, axis=0, keepdims=True)
    s2 = jnp.sum(h * h, axis=0, keepdims=True)
    mu = _fold(s, bsz) / cnt
    m2 = _fold(s2, bsz) / cnt
    var = m2 - mu * mu
    scale = lax.rsqrt(var + EPS) * w[None, :]
    shift = b[None, :] - mu * scale
    return h * _tile(scale, bsz) + _tile(shift, bsz)


def _tc_body(bsz, a_ref, x_ref, tb_ref, w1_ref, b1_ref, w2_ref, b2_ref,
             w3_ref, b3_ref, g1_ref, be1_ref, g2_ref, be2_ref, o_ref):
    a = a_ref[:N, :]
    deg = jnp.sum(a, axis=1, keepdims=True) + 1.0
    dinv = lax.rsqrt(deg)
    A = a[:, :N]

    w1 = w1_ref[...]
    y = jnp.concatenate([_dot(x_ref[g], w1) for g in range(bsz)], axis=1)
    h = _agg(A, y, dinv) + _tile(b1_ref[...][None, :], bsz)
    h = jnp.maximum(h, 0.0)
    h = _bn_nm(h, g1_ref[...], be1_ref[...], bsz)

    w2 = w2_ref[...]
    y = jnp.concatenate(
        [_dot(h[:, g * D_EMB:(g + 1) * D_EMB], w2) for g in range(bsz)], axis=1)
    h = _agg(A, y, dinv) + _tile(b2_ref[...][None, :], bsz)
    h = jnp.maximum(h, 0.0)
    h = _bn_nm(h, g2_ref[...], be2_ref[...], bsz)

    w3 = w3_ref[...]
    y = jnp.concatenate(
        [_dot(h[:, g * D_EMB:(g + 1) * D_EMB], w3) for g in range(bsz)], axis=1)
    h = _agg(A, y, dinv) + _tile(b3_ref[...][None, :], bsz)
    h = jnp.maximum(h, 0.0)
    o_ref[...] = h + (tb_ref[0, 0] - bsz).astype(jnp.float32)


def _tc_forward(a_mat, x3, tb, W1, b1, W2, b2, W3, b3, bn1_w, bn1_b, bn2_w,
                bn2_b, bsz):
    return pl.pallas_call(
        functools.partial(_tc_body, bsz),
        out_shape=jax.ShapeDtypeStruct((N, bsz * D_EMB), jnp.float32),
    )(a_mat, x3, tb, W1, b1, W2, b2, W3, b3, bn1_w, bn1_b, bn2_w, bn2_b)


def kernel(node_list, edge_index, true_batch_size, W1, b1, W2, b2, W3, b3,
           bn1_w, bn1_b, bn2_w, bn2_b):
    bsz, n_per_graph, feat = node_list.shape
    ei_p = jnp.pad(edge_index, ((0, 0), (0, EP - E)))
    a_mat = _sc_build_adj(ei_p).reshape(NR, NC)

    tb = jnp.asarray(true_batch_size, jnp.int32).reshape(1, 1)
    h = _tc_forward(a_mat, node_list, tb, W1, b1, W2, b2, W3, b3,
                    bn1_w, bn1_b, bn2_w, bn2_b, bsz)
    return h.reshape(n_per_graph, bsz, D_EMB).transpose(1, 0, 2)
